# Initial kernel scaffold; baseline (speedup 1.0000x reference)
import jax
import jax.numpy as jnp
from jax.experimental import pallas as pl


def kernel(ind_2, px, Wi, Wj, W0, W1):
    raise NotImplementedError("write your pallas kernel here")


# speedup vs baseline: 1.7957x; 1.7957x over previous
import functools

import jax
import jax.numpy as jnp
from jax import lax
from jax.experimental import pallas as pl
from jax.experimental.pallas import tpu as pltpu
from jax.experimental.pallas import tpu_sc as plsc

N_ATOMS_K = 10000
N_PAIRS_K = 160000
XDIM = 3
N_PROP_K = 128
D = XDIM * N_PROP_K
ROWS = N_ATOMS_K * XDIM

TC_BLK = 3000
TC_GRID = ROWS // TC_BLK

L = 16


def _proj_body(px_ref, wi_ref, wj_ref, w0_ref, w1_ref, yi_ref, yj_ref,
               a_scr, b_scr):
    @pl.when(pl.program_id(0) == 0)
    def _():
        w01 = jnp.dot(w0_ref[...], w1_ref[...],
                      preferred_element_type=jnp.float32,
                      precision=lax.Precision.HIGHEST)
        a_scr[...] = jnp.dot(wi_ref[...], w01,
                             preferred_element_type=jnp.float32,
                             precision=lax.Precision.HIGHEST)
        b_scr[...] = jnp.dot(wj_ref[...], w01,
                             preferred_element_type=jnp.float32,
                             precision=lax.Precision.HIGHEST)

    x = px_ref[...]
    yi_ref[...] = jnp.dot(x, a_scr[...], preferred_element_type=jnp.float32,
                          precision=lax.Precision.HIGHEST)
    yj_ref[...] = jnp.dot(x, b_scr[...], preferred_element_type=jnp.float32,
                          precision=lax.Precision.HIGHEST)


@jax.jit
def _project(px_flat, Wi, Wj, W0, W1):
    wspec = pl.BlockSpec((N_PROP_K, N_PROP_K), lambda i: (0, 0))
    return pl.pallas_call(
        _proj_body,
        grid=(TC_GRID,),
        in_specs=[
            pl.BlockSpec((TC_BLK, N_PROP_K), lambda i: (i, 0)),
            wspec, wspec, wspec, wspec,
        ],
        out_specs=[pl.BlockSpec((TC_BLK, N_PROP_K), lambda i: (i, 0))] * 2,
        out_shape=[jax.ShapeDtypeStruct((ROWS, N_PROP_K), jnp.float32)] * 2,
        scratch_shapes=[
            pltpu.VMEM((N_PROP_K, N_PROP_K), jnp.float32),
            pltpu.VMEM((N_PROP_K, N_PROP_K), jnp.float32),
        ],
    )(px_flat, Wi, Wj, W0, W1)


def _make_sc_gather():
    info = plsc.get_sparse_core_info()
    nc, ns = info.num_cores, info.num_subcores
    nw = nc * ns
    per_w = N_PAIRS_K // nw
    chunk = 64
    full_chunks = per_w // chunk
    rem = per_w - full_chunks * chunk
    n_chunks = full_chunks + (1 if rem else 0)

    mesh = plsc.VectorSubcoreMesh(core_axis_name="c", subcore_axis_name="s")

    @functools.partial(
        pl.kernel,
        mesh=mesh,
        out_type=jax.ShapeDtypeStruct((N_PAIRS_K, D), jnp.float32),
        scratch_types=[
            pltpu.VMEM((n_chunks, chunk), jnp.int32),
            pltpu.VMEM((n_chunks, chunk), jnp.int32),
            pltpu.VMEM((chunk, D), jnp.float32),
            pltpu.VMEM((chunk, D), jnp.float32),
            pltpu.SemaphoreType.DMA,
            pltpu.SemaphoreType.DMA,
        ],
    )
    def sc_gather(yi_hbm, yj_hbm, idxi_hbm, idxj_hbm, out_hbm,
                  idxi_v, idxj_v, buf_a, buf_b, sem_a, sem_b):
        wid = lax.axis_index("s") * nc + lax.axis_index("c")
        base = wid * per_w
        pltpu.sync_copy(idxi_hbm.at[wid], idxi_v)
        pltpu.sync_copy(idxj_hbm.at[wid], idxj_v)

        def do_chunk(c, n_rows, row_off):
            cp_a = pltpu.async_copy(yi_hbm.at[idxi_v.at[c]], buf_a, sem_a)
            cp_b = pltpu.async_copy(yj_hbm.at[idxj_v.at[c]], buf_b, sem_b)
            cp_a.wait()
            cp_b.wait()

            def row_body(r, carry):
                for dd in range(D // L):
                    sl = pl.ds(dd * L, L)
                    buf_a[r, sl] = buf_a[r, sl] + buf_b[r, sl]
                return carry

            lax.fori_loop(0, n_rows, row_body, 0)
            pltpu.sync_copy(buf_a.at[pl.ds(0, n_rows)],
                            out_hbm.at[pl.ds(base + row_off, n_rows)])

        def loop_body(c, carry):
            do_chunk(c, chunk, c * chunk)
            return carry

        lax.fori_loop(0, full_chunks, loop_body, 0)
        if rem:
            do_chunk(full_chunks, rem, full_chunks * chunk)

    return sc_gather, nw, per_w, n_chunks, chunk


def kernel(ind_2, px, Wi, Wj, W0, W1):
    sc_gather, nw, per_w, n_chunks, chunk = _make_sc_gather()

    px_flat = px.reshape(ROWS, N_PROP_K)
    yi, yj = _project(px_flat, Wi, Wj, W0, W1)
    yi_t = yi.reshape(N_ATOMS_K, D)
    yj_t = yj.reshape(N_ATOMS_K, D)

    ind = ind_2.astype(jnp.int32)
    pad = n_chunks * chunk - per_w

    def prep(col):
        a = col.reshape(nw, per_w)
        a = jnp.pad(a, ((0, 0), (0, pad)))
        return a.reshape(nw, n_chunks, chunk)

    idxi = prep(ind[:, 0])
    idxj = prep(ind[:, 1])

    out = sc_gather(yi_t, yj_t, idxi, idxj)
    return out.reshape(N_PAIRS_K, XDIM, N_PROP_K)


# speedup vs baseline: 2.1570x; 1.2012x over previous
import functools

import jax
import jax.numpy as jnp
from jax import lax
from jax.experimental import pallas as pl
from jax.experimental.pallas import tpu as pltpu
from jax.experimental.pallas import tpu_sc as plsc

N_ATOMS_K = 10000
N_PAIRS_K = 160000
XDIM = 3
N_PROP_K = 128
ROWS = N_ATOMS_K * XDIM

TC_BLK = 1000
TC_GRID = N_ATOMS_K // TC_BLK

L = 16


def _proj_body(px_ref, wi_ref, wj_ref, w0_ref, w1_ref, yi_ref, yj_ref,
               a_scr, b_scr):
    @pl.when(pl.program_id(0) == 0)
    def _():
        w01 = jnp.dot(w0_ref[...], w1_ref[...],
                      preferred_element_type=jnp.float32,
                      precision=lax.Precision.HIGHEST)
        a_scr[...] = jnp.dot(wi_ref[...], w01,
                             preferred_element_type=jnp.float32,
                             precision=lax.Precision.HIGHEST)
        b_scr[...] = jnp.dot(wj_ref[...], w01,
                             preferred_element_type=jnp.float32,
                             precision=lax.Precision.HIGHEST)

    x = px_ref[...].reshape(TC_BLK * XDIM, N_PROP_K)
    yi_ref[...] = jnp.dot(
        x, a_scr[...], preferred_element_type=jnp.float32,
        precision=lax.Precision.HIGHEST).reshape(TC_BLK, XDIM, N_PROP_K)
    yj_ref[...] = jnp.dot(
        x, b_scr[...], preferred_element_type=jnp.float32,
        precision=lax.Precision.HIGHEST).reshape(TC_BLK, XDIM, N_PROP_K)


@jax.jit
def _project(px, Wi, Wj, W0, W1):
    wspec = pl.BlockSpec((N_PROP_K, N_PROP_K), lambda i: (0, 0))
    tspec = pl.BlockSpec((TC_BLK, XDIM, N_PROP_K), lambda i: (i, 0, 0))
    return pl.pallas_call(
        _proj_body,
        grid=(TC_GRID,),
        in_specs=[tspec, wspec, wspec, wspec, wspec],
        out_specs=[tspec, tspec],
        out_shape=[jax.ShapeDtypeStruct((N_ATOMS_K, XDIM, N_PROP_K),
                                        jnp.float32)] * 2,
        scratch_shapes=[
            pltpu.VMEM((N_PROP_K, N_PROP_K), jnp.float32),
            pltpu.VMEM((N_PROP_K, N_PROP_K), jnp.float32),
        ],
    )(px, Wi, Wj, W0, W1)


def _make_sc_gather():
    info = plsc.get_sparse_core_info()
    nc, ns = info.num_cores, info.num_subcores
    nw = nc * ns
    per_w = N_PAIRS_K // nw
    chunk = 64
    full_chunks = per_w // chunk
    rem = per_w - full_chunks * chunk
    n_chunks = full_chunks + (1 if rem else 0)

    mesh = plsc.VectorSubcoreMesh(core_axis_name="c", subcore_axis_name="s")

    @functools.partial(
        pl.kernel,
        mesh=mesh,
        out_type=jax.ShapeDtypeStruct((N_PAIRS_K, XDIM, N_PROP_K),
                                      jnp.float32),
        scratch_types=[
            pltpu.VMEM((n_chunks, chunk), jnp.int32),
            pltpu.VMEM((n_chunks, chunk), jnp.int32),
            pltpu.VMEM((chunk, XDIM, N_PROP_K), jnp.float32),
            pltpu.VMEM((chunk, XDIM, N_PROP_K), jnp.float32),
            pltpu.SemaphoreType.DMA,
            pltpu.SemaphoreType.DMA,
        ],
    )
    def sc_gather(yi_hbm, yj_hbm, idxi_hbm, idxj_hbm, out_hbm,
                  idxi_v, idxj_v, buf_a, buf_b, sem_a, sem_b):
        wid = lax.axis_index("s") * nc + lax.axis_index("c")
        base = wid * per_w
        pltpu.sync_copy(idxi_hbm.at[wid], idxi_v)
        pltpu.sync_copy(idxj_hbm.at[wid], idxj_v)

        def do_chunk(c, n_rows, row_off):
            cp_a = pltpu.async_copy(yi_hbm.at[idxi_v.at[c]], buf_a, sem_a)
            cp_b = pltpu.async_copy(yj_hbm.at[idxj_v.at[c]], buf_b, sem_b)
            cp_a.wait()
            cp_b.wait()

            def row_body(r, carry):
                for x in range(XDIM):
                    for dd in range(N_PROP_K // L):
                        sl = pl.ds(dd * L, L)
                        buf_a[r, x, sl] = buf_a[r, x, sl] + buf_b[r, x, sl]
                return carry

            lax.fori_loop(0, n_rows, row_body, 0)
            pltpu.sync_copy(buf_a.at[pl.ds(0, n_rows)],
                            out_hbm.at[pl.ds(base + row_off, n_rows)])

        def loop_body(c, carry):
            do_chunk(c, chunk, c * chunk)
            return carry

        lax.fori_loop(0, full_chunks, loop_body, 0)
        if rem:
            do_chunk(full_chunks, rem, full_chunks * chunk)

    return sc_gather, nw, per_w, n_chunks, chunk


def kernel(ind_2, px, Wi, Wj, W0, W1):
    sc_gather, nw, per_w, n_chunks, chunk = _make_sc_gather()

    yi, yj = _project(px, Wi, Wj, W0, W1)

    ind = ind_2.astype(jnp.int32)
    pad = n_chunks * chunk - per_w

    def prep(col):
        a = col.reshape(nw, per_w)
        a = jnp.pad(a, ((0, 0), (0, pad)))
        return a.reshape(nw, n_chunks, chunk)

    idxi = prep(ind[:, 0])
    idxj = prep(ind[:, 1])

    return sc_gather(yi, yj, idxi, idxj)


# speedup vs baseline: 2.3704x; 1.0989x over previous
import functools

import jax
import jax.numpy as jnp
from jax import lax
from jax.experimental import pallas as pl
from jax.experimental.pallas import tpu as pltpu
from jax.experimental.pallas import tpu_sc as plsc

N_ATOMS_K = 10000
N_PAIRS_K = 160000
XDIM = 3
N_PROP_K = 128
ROWS = N_ATOMS_K * XDIM

TC_BLK = 1000
TC_GRID = N_ATOMS_K // TC_BLK

L = 16


def _proj_body(px_ref, wi_ref, wj_ref, w0_ref, w1_ref, yi_ref, yj_ref,
               a_scr, b_scr):
    @pl.when(pl.program_id(0) == 0)
    def _():
        w01 = jnp.dot(w0_ref[...], w1_ref[...],
                      preferred_element_type=jnp.float32,
                      precision=lax.Precision.HIGHEST)
        a_scr[...] = jnp.dot(wi_ref[...], w01,
                             preferred_element_type=jnp.float32,
                             precision=lax.Precision.HIGHEST)
        b_scr[...] = jnp.dot(wj_ref[...], w01,
                             preferred_element_type=jnp.float32,
                             precision=lax.Precision.HIGHEST)

    x = px_ref[...].reshape(TC_BLK * XDIM, N_PROP_K)
    yi_ref[...] = jnp.dot(
        x, a_scr[...], preferred_element_type=jnp.float32,
        precision=lax.Precision.HIGHEST).reshape(TC_BLK, XDIM, N_PROP_K)
    yj_ref[...] = jnp.dot(
        x, b_scr[...], preferred_element_type=jnp.float32,
        precision=lax.Precision.HIGHEST).reshape(TC_BLK, XDIM, N_PROP_K)


@jax.jit
def _project(px, Wi, Wj, W0, W1):
    wspec = pl.BlockSpec((N_PROP_K, N_PROP_K), lambda i: (0, 0))
    tspec = pl.BlockSpec((TC_BLK, XDIM, N_PROP_K), lambda i: (i, 0, 0))
    return pl.pallas_call(
        _proj_body,
        grid=(TC_GRID,),
        in_specs=[tspec, wspec, wspec, wspec, wspec],
        out_specs=[tspec, tspec],
        out_shape=[jax.ShapeDtypeStruct((N_ATOMS_K, XDIM, N_PROP_K),
                                        jnp.float32)] * 2,
        scratch_shapes=[
            pltpu.VMEM((N_PROP_K, N_PROP_K), jnp.float32),
            pltpu.VMEM((N_PROP_K, N_PROP_K), jnp.float32),
        ],
    )(px, Wi, Wj, W0, W1)


def _make_sc_gather():
    info = plsc.get_sparse_core_info()
    nc, ns = info.num_cores, info.num_subcores
    nw = nc * ns
    per_w = N_PAIRS_K // nw
    chunk = 40
    n_real = per_w // chunk
    n_chunks = n_real + 1

    mesh = plsc.VectorSubcoreMesh(core_axis_name="c", subcore_axis_name="s")

    buf_t = pltpu.VMEM((chunk, XDIM, N_PROP_K), jnp.float32)

    @functools.partial(
        pl.kernel,
        mesh=mesh,
        out_type=jax.ShapeDtypeStruct((N_PAIRS_K, XDIM, N_PROP_K),
                                      jnp.float32),
        scratch_types=[
            pltpu.VMEM((n_chunks, chunk), jnp.int32),
            pltpu.VMEM((n_chunks, chunk), jnp.int32),
            buf_t, buf_t, buf_t, buf_t,
            pltpu.SemaphoreType.DMA, pltpu.SemaphoreType.DMA,
            pltpu.SemaphoreType.DMA, pltpu.SemaphoreType.DMA,
            pltpu.SemaphoreType.DMA, pltpu.SemaphoreType.DMA,
        ],
    )
    def sc_gather(yi_hbm, yj_hbm, idxi_hbm, idxj_hbm, out_hbm,
                  idxi_v, idxj_v, ga0, ga1, gb0, gb1,
                  gsa0, gsa1, gsb0, gsb1, sts0, sts1):
        wid = lax.axis_index("s") * nc + lax.axis_index("c")
        base = wid * per_w
        ga = (ga0, ga1)
        gb = (gb0, gb1)
        gsa = (gsa0, gsa1)
        gsb = (gsb0, gsb1)
        sts = (sts0, sts1)
        pltpu.sync_copy(idxi_hbm.at[wid], idxi_v)
        pltpu.sync_copy(idxj_hbm.at[wid], idxj_v)

        def issue_gather(c, par):
            pltpu.async_copy(yi_hbm.at[idxi_v.at[c]], ga[par], gsa[par])
            pltpu.async_copy(yj_hbm.at[idxj_v.at[c]], gb[par], gsb[par])

        def wait_gather(c, par):
            pltpu.make_async_copy(
                yi_hbm.at[idxi_v.at[c]], ga[par], gsa[par]).wait()
            pltpu.make_async_copy(
                yj_hbm.at[idxj_v.at[c]], gb[par], gsb[par]).wait()

        def wait_store(c, par):
            pltpu.make_async_copy(
                ga[par], out_hbm.at[pl.ds(base + c * chunk, chunk)],
                sts[par]).wait()

        issue_gather(0, 0)

        def step(s, carry):
            for b in range(2):
                c = 2 * s + b
                par = b
                opar = 1 - b

                wait_gather(c, par)

                @pl.when(c <= n_real - 1)
                def _():
                    def row_body(r, cr):
                        for x in range(XDIM):
                            for dd in range(N_PROP_K // L):
                                sl = pl.ds(dd * L, L)
                                ga[par][r, x, sl] = (
                                    ga[par][r, x, sl] + gb[par][r, x, sl])
                        return cr

                    lax.fori_loop(0, chunk, row_body, 0)

                @pl.when(c >= 1)
                def _():
                    wait_store(c - 1, opar)

                @pl.when(c + 1 <= n_chunks - 1)
                def _():
                    issue_gather(c + 1, opar)

                @pl.when(c <= n_real - 1)
                def _():
                    pltpu.async_copy(
                        ga[par],
                        out_hbm.at[pl.ds(base + c * chunk, chunk)],
                        sts[par])
            return carry

        lax.fori_loop(0, n_chunks // 2, step, 0)

    return sc_gather, nw, per_w, n_chunks, chunk


def kernel(ind_2, px, Wi, Wj, W0, W1):
    sc_gather, nw, per_w, n_chunks, chunk = _make_sc_gather()

    yi, yj = _project(px, Wi, Wj, W0, W1)

    ind = ind_2.astype(jnp.int32)
    pad = n_chunks * chunk - per_w

    def prep(col):
        a = col.reshape(nw, per_w)
        a = jnp.pad(a, ((0, 0), (0, pad)))
        return a.reshape(nw, n_chunks, chunk)

    idxi = prep(ind[:, 0])
    idxj = prep(ind[:, 1])

    return sc_gather(yi, yj, idxi, idxj)
